# baseline (speedup 1.0000x reference)
import functools

import jax
import jax.numpy as jnp
from jax import lax
from jax.experimental import pallas as pl
from jax.experimental.pallas import tpu as pltpu
from jax.experimental.pallas import tpu_sc as plsc

HN = 128
HE = 64
NUM_MPN_STEPS = 3

NWORK = 32
CH = 128
BE = 2048
BN = 2000


def _pad_rows(x, rows):
    return jnp.pad(x, ((0, rows - x.shape[0]),) + ((0, 0),) * (x.ndim - 1))


NBUF = 4


@functools.partial(jax.jit, static_argnames=("dim",))
def _sc_gather(table, idx, dim):
    b = idx.shape[0]
    bpw = b // NWORK
    nch = bpw // CH
    ngrp = nch // NBUF + 1
    mesh = plsc.VectorSubcoreMesh(core_axis_name="c", subcore_axis_name="s")

    @functools.partial(
        pl.kernel,
        out_type=jax.ShapeDtypeStruct((b, dim), jnp.float32),
        mesh=mesh,
        scratch_types=[
            pltpu.VMEM((bpw,), jnp.int32),
            [pltpu.VMEM((CH, dim), jnp.float32) for _ in range(NBUF)],
            [pltpu.SemaphoreType.DMA for _ in range(NBUF)],
        ],
    )
    def k(table_hbm, idx_hbm, out_hbm, idx_v, bufs, sems):
        wid = lax.axis_index("s") * 2 + lax.axis_index("c")
        base = wid * bpw
        pltpu.sync_copy(idx_hbm.at[pl.ds(base, bpw)], idx_v)

        def group(g, carry):
            for bi in range(NBUF):
                i = g * NBUF + bi
                @pl.when(jnp.logical_and(i - 2 >= 0, i - 2 < nch))
                def _():
                    pltpu.make_async_copy(
                        bufs[(bi - 2) % NBUF],
                        out_hbm.at[pl.ds(base + (i - 2) * CH, CH)],
                        sems[(bi - 2) % NBUF]).wait()
                @pl.when(i < nch)
                def _():
                    pltpu.async_copy(
                        table_hbm.at[idx_v.at[pl.ds(i * CH, CH)]],
                        bufs[bi], sems[bi])
                @pl.when(jnp.logical_and(i - 1 >= 0, i - 1 < nch))
                def _():
                    bj = (bi - 1) % NBUF
                    pltpu.make_async_copy(
                        table_hbm.at[idx_v.at[pl.ds((i - 1) * CH, CH)]],
                        bufs[bj], sems[bj]).wait()
                    pltpu.async_copy(
                        bufs[bj],
                        out_hbm.at[pl.ds(base + (i - 1) * CH, CH)],
                        sems[bj])
            return carry

        lax.fori_loop(0, ngrp, group, 0)

    return k(table, idx)


@jax.jit
def _sc_gather_add2(qs, qd, src, dst):
    b = src.shape[0]
    bpw = b // NWORK
    nch = bpw // CH
    mesh = plsc.VectorSubcoreMesh(core_axis_name="c", subcore_axis_name="s")

    @functools.partial(
        pl.kernel,
        out_type=jax.ShapeDtypeStruct((b, HN), jnp.float32),
        mesh=mesh,
        scratch_types=[
            pltpu.VMEM((bpw,), jnp.int32),
            pltpu.VMEM((bpw,), jnp.int32),
            [pltpu.VMEM((CH, HN), jnp.float32) for _ in range(NBUF)],
            [pltpu.SemaphoreType.DMA for _ in range(NBUF)],
        ],
    )
    def k(qs_hbm, qd_hbm, src_hbm, dst_hbm, out_hbm, sidx_v, didx_v, bufs, sems):
        wid = lax.axis_index("s") * 2 + lax.axis_index("c")
        base = wid * bpw
        pltpu.sync_copy(src_hbm.at[pl.ds(base, bpw)], sidx_v)
        pltpu.sync_copy(dst_hbm.at[pl.ds(base, bpw)], didx_v)

        def group(g, carry):
            for bi in range(NBUF):
                i = g * NBUF + bi
                @pl.when(jnp.logical_and(i - 3 >= 0, i - 3 < nch))
                def _():
                    bj = (bi - 3) % NBUF
                    pltpu.make_async_copy(
                        bufs[bj],
                        out_hbm.at[pl.ds(base + (i - 3) * CH, CH)],
                        sems[bj]).wait()
                @pl.when(i < nch)
                def _():
                    pltpu.async_copy(
                        qs_hbm.at[sidx_v.at[pl.ds(i * CH, CH)]],
                        bufs[bi], sems[bi])
                @pl.when(jnp.logical_and(i - 1 >= 0, i - 1 < nch))
                def _():
                    bj = (bi - 1) % NBUF
                    pltpu.make_async_copy(
                        qs_hbm.at[sidx_v.at[pl.ds((i - 1) * CH, CH)]],
                        bufs[bj], sems[bj]).wait()
                    pltpu.async_copy(
                        qd_hbm.at[didx_v.at[pl.ds((i - 1) * CH, CH)]],
                        bufs[bj], sems[bj], add=True)
                @pl.when(jnp.logical_and(i - 2 >= 0, i - 2 < nch))
                def _():
                    bj = (bi - 2) % NBUF
                    pltpu.make_async_copy(
                        qd_hbm.at[didx_v.at[pl.ds((i - 2) * CH, CH)]],
                        bufs[bj], sems[bj]).wait()
                    pltpu.async_copy(
                        bufs[bj],
                        out_hbm.at[pl.ds(base + (i - 2) * CH, CH)],
                        sems[bj])
            return carry

        lax.fori_loop(0, nch // NBUF + 1, group, 0)

    return k(qs, qd, src, dst)


@functools.partial(jax.jit, static_argnames=("num_nodes",))
def _sc_segment_sum(msg, dst, zeros, num_nodes):
    e_rows = msg.shape[0]
    quarter = num_nodes // 4
    rpt = (quarter // 16) // 8 * 8
    tail = quarter - 16 * rpt
    garbage = quarter
    sh_rows = quarter + 8
    rows_t = e_rows // 16
    nch = rows_t // CH
    mesh = plsc.VectorSubcoreMesh(core_axis_name="c", subcore_axis_name="s")

    @functools.partial(
        pl.kernel,
        out_type=jax.ShapeDtypeStruct((num_nodes, HN), jnp.float32),
        mesh=mesh,
        scratch_types=[
            pltpu.VMEM((rows_t,), jnp.int32),
            [pltpu.VMEM((CH,), jnp.int32) for _ in range(NBUF)],
            [pltpu.VMEM((CH, HN), jnp.float32) for _ in range(NBUF)],
            pltpu.VMEM_SHARED((sh_rows, HN), jnp.float32),
            [pltpu.SemaphoreType.DMA for _ in range(NBUF)],
        ],
    )
    def k(msg_hbm, dst_hbm, zeros_hbm, out_hbm, didx_v, lidxs, bufs, acc_sh, sems):
        c = lax.axis_index("c")
        s = lax.axis_index("s")
        tbase = s * rows_t
        pltpu.sync_copy(dst_hbm.at[pl.ds(tbase, rows_t)], didx_v)

        for p in range(2):
            node_base = c * 2 * quarter + p * quarter
            pltpu.sync_copy(zeros_hbm.at[pl.ds(s * rpt, rpt)],
                            acc_sh.at[pl.ds(s * rpt, rpt)])
            if tail:
                @pl.when(s == 0)
                def _():
                    pltpu.sync_copy(zeros_hbm.at[pl.ds(16 * rpt, tail)],
                                    acc_sh.at[pl.ds(16 * rpt, tail)])
            plsc.subcore_barrier()

            def group(g, carry):
                for bi in range(NBUF):
                    i = g * NBUF + bi
                    @pl.when(jnp.logical_and(i - 2 >= 0, i - 2 < nch))
                    def _():
                        bj = (bi - 2) % NBUF
                        pltpu.make_async_copy(
                            bufs[bj], acc_sh.at[lidxs[bj]], sems[bj]).wait()
                    @pl.when(i < nch)
                    def _():
                        pltpu.async_copy(
                            msg_hbm.at[pl.ds(tbase + i * CH, CH)],
                            bufs[bi], sems[bi])
                    @pl.when(jnp.logical_and(i - 1 >= 0, i - 1 < nch))
                    def _():
                        bj = (bi - 1) % NBUF
                        for j in range(CH // 16):
                            v = didx_v[pl.ds((i - 1) * CH + j * 16, 16)]
                            lo = v - node_base
                            ok = (lo >= 0) & (lo < quarter)
                            lidxs[bj][pl.ds(j * 16, 16)] = jnp.where(ok, lo, garbage)
                        pltpu.make_async_copy(
                            msg_hbm.at[pl.ds(tbase + (i - 1) * CH, CH)],
                            bufs[bj], sems[bj]).wait()
                        pltpu.async_copy(
                            bufs[bj], acc_sh.at[lidxs[bj]], sems[bj], add=True)
                return carry

            lax.fori_loop(0, nch // NBUF + 1, group, 0)
            plsc.subcore_barrier()
            pltpu.sync_copy(acc_sh.at[pl.ds(s * rpt, rpt)],
                            out_hbm.at[pl.ds(node_base + s * rpt, rpt)])
            if tail:
                @pl.when(s == 0)
                def _():
                    pltpu.sync_copy(
                        acc_sh.at[pl.ds(16 * rpt, tail)],
                        out_hbm.at[pl.ds(node_base + 16 * rpt, tail)])
            plsc.subcore_barrier()

    return k(msg, dst, zeros)


def _full(shape):
    return pl.BlockSpec(shape, lambda i: (0,) * len(shape))


def _rows(block, width):
    return pl.BlockSpec((block, width), lambda i: (i, 0))


def _dot(a, b):
    return jnp.dot(a, b, preferred_element_type=jnp.float32)


def _tc_node_init(nf, wn, bn, wm_h):
    n = nf.shape[0]

    def body(nf_ref, wn_ref, bn_ref, wmh_ref, h_ref, p_ref):
        h = jnp.maximum(_dot(nf_ref[...], wn_ref[...]) + bn_ref[...], 0.0)
        h_ref[...] = h
        p_ref[...] = _dot(h, wmh_ref[...])

    return pl.pallas_call(
        body,
        grid=(n // BN,),
        in_specs=[_rows(BN, HN), _full((HN, HN)), _full((1, HN)), _full((HN, HN))],
        out_specs=[_rows(BN, HN), _rows(BN, HN)],
        out_shape=[jax.ShapeDtypeStruct((n, HN), jnp.float32)] * 2,
    )(nf, wn, bn, wm_h)


def _tc_edge_init(ef, we, be):
    e_rows = ef.shape[0]

    def body(ef_ref, we_ref, be_ref, out_ref):
        out_ref[...] = jnp.maximum(ef_ref[...] * we_ref[...] + be_ref[...], 0.0)

    return pl.pallas_call(
        body,
        grid=(e_rows // BE,),
        in_specs=[_rows(BE, 1), _full((1, HE)), _full((1, HE))],
        out_specs=_rows(BE, HE),
        out_shape=jax.ShapeDtypeStruct((e_rows, HE), jnp.float32),
    )(ef, we, be)


def _tc_msg(g, e, wm_e, bm):
    e_rows = g.shape[0]

    def body(g_ref, e_ref, w_ref, b_ref, out_ref):
        out_ref[...] = jnp.maximum(
            g_ref[...] + _dot(e_ref[...], w_ref[...]) + b_ref[...], 0.0)

    return pl.pallas_call(
        body,
        grid=(e_rows // BE,),
        in_specs=[_rows(BE, HN), _rows(BE, HE), _full((HE, HN)), _full((1, HN))],
        out_specs=_rows(BE, HN),
        out_shape=jax.ShapeDtypeStruct((e_rows, HN), jnp.float32),
    )(g, e, wm_e, bm)


def _tc_node_update(h, agg, wu_h, wu_a, bu, wue_s, wue_d, wm_h):
    n = h.shape[0]

    def body(h_ref, agg_ref, wuh_ref, wua_ref, bu_ref, ws_ref, wd_ref,
             wmh_ref, hn_ref, qs_ref, qd_ref, p_ref):
        hn = jnp.maximum(
            _dot(h_ref[...], wuh_ref[...]) + _dot(agg_ref[...], wua_ref[...])
            + bu_ref[...], 0.0)
        hn_ref[...] = hn
        pad = jnp.zeros((hn.shape[0], HN - HE), jnp.float32)
        qs_ref[...] = jnp.concatenate([_dot(hn, ws_ref[...]), pad], axis=1)
        qd_ref[...] = jnp.concatenate([_dot(hn, wd_ref[...]), pad], axis=1)
        p_ref[...] = _dot(hn, wmh_ref[...])

    return pl.pallas_call(
        body,
        grid=(n // BN,),
        in_specs=[_rows(BN, HN), _rows(BN, HN), _full((HN, HN)), _full((HN, HN)),
                  _full((1, HN)), _full((HN, HE)), _full((HN, HE)), _full((HN, HN))],
        out_specs=[_rows(BN, HN), _rows(BN, HN), _rows(BN, HN), _rows(BN, HN)],
        out_shape=[jax.ShapeDtypeStruct((n, HN), jnp.float32),
                   jax.ShapeDtypeStruct((n, HN), jnp.float32),
                   jax.ShapeDtypeStruct((n, HN), jnp.float32),
                   jax.ShapeDtypeStruct((n, HN), jnp.float32)],
    )(h, agg, wu_h, wu_a, bu, wue_s, wue_d, wm_h)


def _tc_edge_update(sd, e, wue_e, bue):
    e_rows = sd.shape[0]

    def body(sd_ref, e_ref, w_ref, b_ref, out_ref):
        out_ref[...] = jnp.maximum(
            sd_ref[:, :HE] + _dot(e_ref[...], w_ref[...]) + b_ref[...], 0.0)

    return pl.pallas_call(
        body,
        grid=(e_rows // BE,),
        in_specs=[_rows(BE, HN), _rows(BE, HE), _full((HE, HE)), _full((1, HE))],
        out_specs=_rows(BE, HE),
        out_shape=jax.ShapeDtypeStruct((e_rows, HE), jnp.float32),
    )(sd, e, wue_e, bue)


def _tc_classifier(ha, hb, e1, e2, w1a, w1b, w1p, b1, w2, b2, w3, b3,
                   w4, b4, w5, b5, w6, b6):
    n = ha.shape[0]

    def body(ha_ref, hb_ref, e1_ref, e2_ref, w1a_ref, w1b_ref, w1p_ref, b1_ref,
             w2_ref, b2_ref, w3_ref, b3_ref, w4_ref, b4_ref, w5_ref, b5_ref,
             w6_ref, b6_ref, out_ref):
        pbh = e1_ref[...] + e2_ref[...]
        x = jnp.maximum(
            _dot(ha_ref[...], w1a_ref[...]) + _dot(hb_ref[...], w1b_ref[...])
            + _dot(pbh, w1p_ref[...]) + b1_ref[...], 0.0)
        x = jnp.maximum(_dot(x, w2_ref[...]) + b2_ref[...], 0.0)
        x = jnp.maximum(_dot(x, w3_ref[...]) + b3_ref[...], 0.0)
        x = jnp.maximum(_dot(x, w4_ref[...]) + b4_ref[...], 0.0)
        x = jnp.maximum(_dot(x, w5_ref[...]) + b5_ref[...], 0.0)
        z = _dot(x, w6_ref[...]) + b6_ref[...]
        m = jnp.max(z, axis=1, keepdims=True)
        ez = jnp.exp(z - m)
        out_ref[...] = ez / jnp.sum(ez, axis=1, keepdims=True)

    return pl.pallas_call(
        body,
        grid=(n // BN,),
        in_specs=[_rows(BN, HN), _rows(BN, HN), _rows(BN, HE), _rows(BN, HE),
                  _full((HN, 128)), _full((HN, 128)), _full((HE, 128)),
                  _full((1, 128)),
                  _full((128, 256)), _full((1, 256)),
                  _full((256, 256)), _full((1, 256)),
                  _full((256, 128)), _full((1, 128)),
                  _full((128, 64)), _full((1, 64)),
                  _full((64, 4)), _full((1, 4))],
        out_specs=_rows(BN, 4),
        out_shape=jax.ShapeDtypeStruct((n, 4), jnp.float32),
    )(ha, hb, e1, e2, w1a, w1b, w1p, b1, w2, b2, w3, b3, w4, b4, w5, b5, w6, b6)


def kernel(a_node_features, a_edge_features, a_edges, a_batch_indices,
           b_node_features, b_edge_features, b_edges, b_batch_indices,
           proposed_bonds, Wn, bn, We, be, Wm, bm, Wu, bu, Wue, bue,
           W1, b1, W2, b2, W3, b3, W4, b4, W5, b5, W6, b6):
    num_a = a_node_features.shape[0]
    ea = a_edges.shape[1]
    eb = b_edges.shape[1]
    nc = proposed_bonds.shape[1]
    num_nodes = num_a + b_node_features.shape[0]
    e_rows = ea + eb + 2 * nc
    align = NWORK * CH
    e_pad = -(-e_rows // align) * align
    cls_pad = -(-2 * nc // align) * align

    pb0 = proposed_bonds[0]
    pb1 = proposed_bonds[1] + num_a
    add_edges = jnp.concatenate(
        [jnp.stack([pb0, pb1]), jnp.stack([pb1, pb0])], axis=1)
    edges = jnp.concatenate([a_edges, b_edges + num_a, add_edges], axis=1)
    src = jnp.pad(edges[0], (0, e_pad - e_rows))
    dst_g = jnp.pad(edges[1], (0, e_pad - e_rows))
    dst_s = jnp.pad(edges[1], (0, e_pad - e_rows),
                    constant_values=num_nodes)
    nf = jnp.concatenate([a_node_features, b_node_features], axis=0)
    ef = jnp.concatenate(
        [a_edge_features, b_edge_features,
         jnp.full((2 * nc, 1), -1000.0, jnp.float32)], axis=0)
    ef = _pad_rows(ef, e_pad)
    idx_cls = jnp.pad(jnp.concatenate([pb0, pb1]), (0, cls_pad - 2 * nc))
    zeros_half = jnp.zeros((num_nodes // 4, HN), jnp.float32)

    wm_h, wm_e = Wm[:HN], Wm[HN:]
    wu_h, wu_a = Wu[:HN], Wu[HN:]
    wue_s, wue_d, wue_e = Wue[:HN], Wue[HN:2 * HN], Wue[2 * HN:]
    w1a, w1b, w1p = W1[:HN], W1[HN:2 * HN], W1[2 * HN:]
    bn2, bm2, bu2, be2, bue2 = (x.reshape(1, -1) for x in (bn, bm, bu, be, bue))
    b1r, b2r, b3r, b4r, b5r, b6r = (x.reshape(1, -1)
                                    for x in (b1, b2, b3, b4, b5, b6))

    h, p = _tc_node_init(nf, Wn, bn2, wm_h)
    e = _tc_edge_init(ef, We, be2)
    for _ in range(NUM_MPN_STEPS):
        g = _sc_gather(p, src, dim=HN)
        msg = _tc_msg(g, e, wm_e, bm2)
        agg = _sc_segment_sum(msg, dst_s, zeros_half, num_nodes)
        h, qs, qd, p = _tc_node_update(h, agg, wu_h, wu_a, bu2,
                                       wue_s, wue_d, wm_h)
        sd = _sc_gather_add2(qs, qd, src, dst_g)
        e = _tc_edge_update(sd, e, wue_e, bue2)

    hh = _sc_gather(h, idx_cls, dim=HN)
    ha, hb = hh[:nc], hh[nc:2 * nc]
    off = ea + eb
    e1 = e[off:off + nc]
    e2 = e[off + nc:off + 2 * nc]
    return _tc_classifier(ha, hb, e1, e2, w1a, w1b, w1p, b1r,
                          W2, b2r, W3, b3r, W4, b4r, W5, b5r, W6, b6r)


# speedup vs baseline: 1.3254x; 1.3254x over previous
import functools

import jax
import jax.numpy as jnp
from jax import lax
from jax.experimental import pallas as pl
from jax.experimental.pallas import tpu as pltpu
from jax.experimental.pallas import tpu_sc as plsc

HN = 128
HE = 64
NUM_MPN_STEPS = 3

NWORK = 32
CH = 128
BE = 2048
BN = 2000


def _pad_rows(x, rows):
    return jnp.pad(x, ((0, rows - x.shape[0]),) + ((0, 0),) * (x.ndim - 1))


NBUF = 4


@functools.partial(jax.jit, static_argnames=("dim",))
def _sc_gather(table, idx, dim):
    b = idx.shape[0]
    bpw = b // NWORK
    nch = bpw // CH
    ngrp = nch // NBUF + 1
    mesh = plsc.VectorSubcoreMesh(core_axis_name="c", subcore_axis_name="s")

    @functools.partial(
        pl.kernel,
        out_type=jax.ShapeDtypeStruct((b, dim), jnp.float32),
        mesh=mesh,
        scratch_types=[
            pltpu.VMEM((bpw,), jnp.int32),
            [pltpu.VMEM((CH, dim), jnp.float32) for _ in range(NBUF)],
            [pltpu.SemaphoreType.DMA for _ in range(NBUF)],
        ],
    )
    def k(table_hbm, idx_hbm, out_hbm, idx_v, bufs, sems):
        wid = lax.axis_index("s") * 2 + lax.axis_index("c")
        base = wid * bpw
        pltpu.sync_copy(idx_hbm.at[pl.ds(base, bpw)], idx_v)

        def group(g, carry):
            for bi in range(NBUF):
                i = g * NBUF + bi
                @pl.when(jnp.logical_and(i - 2 >= 0, i - 2 < nch))
                def _():
                    pltpu.make_async_copy(
                        bufs[(bi - 2) % NBUF],
                        out_hbm.at[pl.ds(base + (i - 2) * CH, CH)],
                        sems[(bi - 2) % NBUF]).wait()
                @pl.when(i < nch)
                def _():
                    pltpu.async_copy(
                        table_hbm.at[idx_v.at[pl.ds(i * CH, CH)]],
                        bufs[bi], sems[bi])
                @pl.when(jnp.logical_and(i - 1 >= 0, i - 1 < nch))
                def _():
                    bj = (bi - 1) % NBUF
                    pltpu.make_async_copy(
                        table_hbm.at[idx_v.at[pl.ds((i - 1) * CH, CH)]],
                        bufs[bj], sems[bj]).wait()
                    pltpu.async_copy(
                        bufs[bj],
                        out_hbm.at[pl.ds(base + (i - 1) * CH, CH)],
                        sems[bj])
            return carry

        lax.fori_loop(0, ngrp, group, 0)

    return k(table, idx)


@jax.jit
def _sc_gather_add2(qs, qd, src, dst):
    b = src.shape[0]
    bpw = b // NWORK
    nch = bpw // CH
    mesh = plsc.VectorSubcoreMesh(core_axis_name="c", subcore_axis_name="s")

    @functools.partial(
        pl.kernel,
        out_type=jax.ShapeDtypeStruct((b, HN), jnp.float32),
        mesh=mesh,
        scratch_types=[
            pltpu.VMEM((bpw,), jnp.int32),
            pltpu.VMEM((bpw,), jnp.int32),
            [pltpu.VMEM((CH, HN), jnp.float32) for _ in range(NBUF)],
            [pltpu.SemaphoreType.DMA for _ in range(NBUF)],
        ],
    )
    def k(qs_hbm, qd_hbm, src_hbm, dst_hbm, out_hbm, sidx_v, didx_v, bufs, sems):
        wid = lax.axis_index("s") * 2 + lax.axis_index("c")
        base = wid * bpw
        pltpu.sync_copy(src_hbm.at[pl.ds(base, bpw)], sidx_v)
        pltpu.sync_copy(dst_hbm.at[pl.ds(base, bpw)], didx_v)

        def group(g, carry):
            for bi in range(NBUF):
                i = g * NBUF + bi
                @pl.when(jnp.logical_and(i - 3 >= 0, i - 3 < nch))
                def _():
                    bj = (bi - 3) % NBUF
                    pltpu.make_async_copy(
                        bufs[bj],
                        out_hbm.at[pl.ds(base + (i - 3) * CH, CH)],
                        sems[bj]).wait()
                @pl.when(i < nch)
                def _():
                    pltpu.async_copy(
                        qs_hbm.at[sidx_v.at[pl.ds(i * CH, CH)]],
                        bufs[bi], sems[bi])
                @pl.when(jnp.logical_and(i - 1 >= 0, i - 1 < nch))
                def _():
                    bj = (bi - 1) % NBUF
                    pltpu.make_async_copy(
                        qs_hbm.at[sidx_v.at[pl.ds((i - 1) * CH, CH)]],
                        bufs[bj], sems[bj]).wait()
                    pltpu.async_copy(
                        qd_hbm.at[didx_v.at[pl.ds((i - 1) * CH, CH)]],
                        bufs[bj], sems[bj], add=True)
                @pl.when(jnp.logical_and(i - 2 >= 0, i - 2 < nch))
                def _():
                    bj = (bi - 2) % NBUF
                    pltpu.make_async_copy(
                        qd_hbm.at[didx_v.at[pl.ds((i - 2) * CH, CH)]],
                        bufs[bj], sems[bj]).wait()
                    pltpu.async_copy(
                        bufs[bj],
                        out_hbm.at[pl.ds(base + (i - 2) * CH, CH)],
                        sems[bj])
            return carry

        lax.fori_loop(0, nch // NBUF + 1, group, 0)

    return k(qs, qd, src, dst)


@functools.partial(jax.jit, static_argnames=("num_nodes",))
def _sc_segment_sum(msg, dst, zeros, num_nodes):
    e_rows = msg.shape[0]
    half = num_nodes // 2
    rpt = (half // 16) // 8 * 8
    tail = half - 16 * rpt
    garbage = half
    sh_rows = half + 8
    rows_t = e_rows // 16
    NB2 = 2
    chs = CH
    nch = rows_t // chs
    mesh = plsc.VectorSubcoreMesh(core_axis_name="c", subcore_axis_name="s")

    @functools.partial(
        pl.kernel,
        out_type=jax.ShapeDtypeStruct((num_nodes, HN), jnp.float32),
        mesh=mesh,
        scratch_types=[
            [pltpu.VMEM((chs,), jnp.int32) for _ in range(NB2)],
            [pltpu.VMEM((chs,), jnp.int32) for _ in range(NB2)],
            [pltpu.VMEM((chs, HN), jnp.float32) for _ in range(NB2)],
            pltpu.VMEM_SHARED((sh_rows, HN), jnp.float32),
            [pltpu.SemaphoreType.DMA for _ in range(NB2)],
            [pltpu.SemaphoreType.DMA for _ in range(NB2)],
        ],
    )
    def k(msg_hbm, dst_hbm, zeros_hbm, out_hbm, didxs, lidxs, bufs, acc_sh,
          sems, dsems):
        c = lax.axis_index("c")
        s = lax.axis_index("s")
        node_base = c * half
        tbase = s * rows_t
        pltpu.sync_copy(zeros_hbm.at[pl.ds(s * rpt, rpt)],
                        acc_sh.at[pl.ds(s * rpt, rpt)])
        if tail:
            @pl.when(s == 0)
            def _():
                pltpu.sync_copy(zeros_hbm.at[pl.ds(16 * rpt, tail)],
                                acc_sh.at[pl.ds(16 * rpt, tail)])
        plsc.subcore_barrier()

        def group(g, carry):
            for bi in range(NB2):
                i = g * NB2 + bi
                @pl.when(i < nch)
                def _():
                    pltpu.async_copy(
                        dst_hbm.at[pl.ds(tbase + i * chs, chs)],
                        didxs[bi], dsems[bi])
                    pltpu.async_copy(
                        msg_hbm.at[pl.ds(tbase + i * chs, chs)],
                        bufs[bi], sems[bi])
                @pl.when(jnp.logical_and(i - 1 >= 0, i - 1 < nch))
                def _():
                    bj = (bi - 1) % NB2
                    pltpu.make_async_copy(
                        dst_hbm.at[pl.ds(tbase + (i - 1) * chs, chs)],
                        didxs[bj], dsems[bj]).wait()
                    pltpu.make_async_copy(
                        msg_hbm.at[pl.ds(tbase + (i - 1) * chs, chs)],
                        bufs[bj], sems[bj]).wait()
                    for j in range(chs // 16):
                        v = didxs[bj][pl.ds(j * 16, 16)]
                        lo = v - node_base
                        ok = (lo >= 0) & (lo < half)
                        lidxs[bj][pl.ds(j * 16, 16)] = jnp.where(ok, lo, garbage)
                    pltpu.sync_copy(bufs[bj], acc_sh.at[lidxs[bj]], add=True)
            return carry

        lax.fori_loop(0, nch // NB2 + 1, group, 0)
        plsc.subcore_barrier()
        pltpu.sync_copy(acc_sh.at[pl.ds(s * rpt, rpt)],
                        out_hbm.at[pl.ds(node_base + s * rpt, rpt)])
        if tail:
            @pl.when(s == 0)
            def _():
                pltpu.sync_copy(
                    acc_sh.at[pl.ds(16 * rpt, tail)],
                    out_hbm.at[pl.ds(node_base + 16 * rpt, tail)])

    return k(msg, dst, zeros)


def _full(shape):
    return pl.BlockSpec(shape, lambda i: (0,) * len(shape))


def _rows(block, width):
    return pl.BlockSpec((block, width), lambda i: (i, 0))


def _dot(a, b):
    return jnp.dot(a, b, preferred_element_type=jnp.float32)


def _tc_node_init(nf, wn, bn, wm_h):
    n = nf.shape[0]

    def body(nf_ref, wn_ref, bn_ref, wmh_ref, h_ref, p_ref):
        h = jnp.maximum(_dot(nf_ref[...], wn_ref[...]) + bn_ref[...], 0.0)
        h_ref[...] = h
        p_ref[...] = _dot(h, wmh_ref[...])

    return pl.pallas_call(
        body,
        grid=(n // BN,),
        in_specs=[_rows(BN, HN), _full((HN, HN)), _full((1, HN)), _full((HN, HN))],
        out_specs=[_rows(BN, HN), _rows(BN, HN)],
        out_shape=[jax.ShapeDtypeStruct((n, HN), jnp.float32)] * 2,
    )(nf, wn, bn, wm_h)


def _tc_edge_init(ef, we, be):
    e_rows = ef.shape[0]

    def body(ef_ref, we_ref, be_ref, out_ref):
        out_ref[...] = jnp.maximum(ef_ref[...] * we_ref[...] + be_ref[...], 0.0)

    return pl.pallas_call(
        body,
        grid=(e_rows // BE,),
        in_specs=[_rows(BE, 1), _full((1, HE)), _full((1, HE))],
        out_specs=_rows(BE, HE),
        out_shape=jax.ShapeDtypeStruct((e_rows, HE), jnp.float32),
    )(ef, we, be)


def _tc_msg(g, e, wm_e, bm):
    e_rows = g.shape[0]

    def body(g_ref, e_ref, w_ref, b_ref, out_ref):
        out_ref[...] = jnp.maximum(
            g_ref[...] + _dot(e_ref[...], w_ref[...]) + b_ref[...], 0.0)

    return pl.pallas_call(
        body,
        grid=(e_rows // BE,),
        in_specs=[_rows(BE, HN), _rows(BE, HE), _full((HE, HN)), _full((1, HN))],
        out_specs=_rows(BE, HN),
        out_shape=jax.ShapeDtypeStruct((e_rows, HN), jnp.float32),
    )(g, e, wm_e, bm)


def _tc_node_update(h, agg, wu_h, wu_a, bu, wue_s, wue_d, wm_h):
    n = h.shape[0]

    def body(h_ref, agg_ref, wuh_ref, wua_ref, bu_ref, ws_ref, wd_ref,
             wmh_ref, hn_ref, qs_ref, qd_ref, p_ref):
        hn = jnp.maximum(
            _dot(h_ref[...], wuh_ref[...]) + _dot(agg_ref[...], wua_ref[...])
            + bu_ref[...], 0.0)
        hn_ref[...] = hn
        pad = jnp.zeros((hn.shape[0], HN - HE), jnp.float32)
        qs_ref[...] = jnp.concatenate([_dot(hn, ws_ref[...]), pad], axis=1)
        qd_ref[...] = jnp.concatenate([_dot(hn, wd_ref[...]), pad], axis=1)
        p_ref[...] = _dot(hn, wmh_ref[...])

    return pl.pallas_call(
        body,
        grid=(n // BN,),
        in_specs=[_rows(BN, HN), _rows(BN, HN), _full((HN, HN)), _full((HN, HN)),
                  _full((1, HN)), _full((HN, HE)), _full((HN, HE)), _full((HN, HN))],
        out_specs=[_rows(BN, HN), _rows(BN, HN), _rows(BN, HN), _rows(BN, HN)],
        out_shape=[jax.ShapeDtypeStruct((n, HN), jnp.float32),
                   jax.ShapeDtypeStruct((n, HN), jnp.float32),
                   jax.ShapeDtypeStruct((n, HN), jnp.float32),
                   jax.ShapeDtypeStruct((n, HN), jnp.float32)],
    )(h, agg, wu_h, wu_a, bu, wue_s, wue_d, wm_h)


def _tc_edge_update(sd, e, wue_e, bue):
    e_rows = sd.shape[0]

    def body(sd_ref, e_ref, w_ref, b_ref, out_ref):
        out_ref[...] = jnp.maximum(
            sd_ref[:, :HE] + _dot(e_ref[...], w_ref[...]) + b_ref[...], 0.0)

    return pl.pallas_call(
        body,
        grid=(e_rows // BE,),
        in_specs=[_rows(BE, HN), _rows(BE, HE), _full((HE, HE)), _full((1, HE))],
        out_specs=_rows(BE, HE),
        out_shape=jax.ShapeDtypeStruct((e_rows, HE), jnp.float32),
    )(sd, e, wue_e, bue)


def _tc_classifier(ha, hb, e1, e2, w1a, w1b, w1p, b1, w2, b2, w3, b3,
                   w4, b4, w5, b5, w6, b6):
    n = ha.shape[0]

    def body(ha_ref, hb_ref, e1_ref, e2_ref, w1a_ref, w1b_ref, w1p_ref, b1_ref,
             w2_ref, b2_ref, w3_ref, b3_ref, w4_ref, b4_ref, w5_ref, b5_ref,
             w6_ref, b6_ref, out_ref):
        pbh = e1_ref[...] + e2_ref[...]
        x = jnp.maximum(
            _dot(ha_ref[...], w1a_ref[...]) + _dot(hb_ref[...], w1b_ref[...])
            + _dot(pbh, w1p_ref[...]) + b1_ref[...], 0.0)
        x = jnp.maximum(_dot(x, w2_ref[...]) + b2_ref[...], 0.0)
        x = jnp.maximum(_dot(x, w3_ref[...]) + b3_ref[...], 0.0)
        x = jnp.maximum(_dot(x, w4_ref[...]) + b4_ref[...], 0.0)
        x = jnp.maximum(_dot(x, w5_ref[...]) + b5_ref[...], 0.0)
        z = _dot(x, w6_ref[...]) + b6_ref[...]
        m = jnp.max(z, axis=1, keepdims=True)
        ez = jnp.exp(z - m)
        out_ref[...] = ez / jnp.sum(ez, axis=1, keepdims=True)

    return pl.pallas_call(
        body,
        grid=(n // BN,),
        in_specs=[_rows(BN, HN), _rows(BN, HN), _rows(BN, HE), _rows(BN, HE),
                  _full((HN, 128)), _full((HN, 128)), _full((HE, 128)),
                  _full((1, 128)),
                  _full((128, 256)), _full((1, 256)),
                  _full((256, 256)), _full((1, 256)),
                  _full((256, 128)), _full((1, 128)),
                  _full((128, 64)), _full((1, 64)),
                  _full((64, 4)), _full((1, 4))],
        out_specs=_rows(BN, 4),
        out_shape=jax.ShapeDtypeStruct((n, 4), jnp.float32),
    )(ha, hb, e1, e2, w1a, w1b, w1p, b1, w2, b2, w3, b3, w4, b4, w5, b5, w6, b6)


def kernel(a_node_features, a_edge_features, a_edges, a_batch_indices,
           b_node_features, b_edge_features, b_edges, b_batch_indices,
           proposed_bonds, Wn, bn, We, be, Wm, bm, Wu, bu, Wue, bue,
           W1, b1, W2, b2, W3, b3, W4, b4, W5, b5, W6, b6):
    num_a = a_node_features.shape[0]
    ea = a_edges.shape[1]
    eb = b_edges.shape[1]
    nc = proposed_bonds.shape[1]
    num_nodes = num_a + b_node_features.shape[0]
    e_rows = ea + eb + 2 * nc
    align = NWORK * CH
    e_pad = -(-e_rows // align) * align
    cls_pad = -(-2 * nc // align) * align

    pb0 = proposed_bonds[0]
    pb1 = proposed_bonds[1] + num_a
    add_edges = jnp.concatenate(
        [jnp.stack([pb0, pb1]), jnp.stack([pb1, pb0])], axis=1)
    edges = jnp.concatenate([a_edges, b_edges + num_a, add_edges], axis=1)
    src = jnp.pad(edges[0], (0, e_pad - e_rows))
    dst_g = jnp.pad(edges[1], (0, e_pad - e_rows))
    dst_s = jnp.pad(edges[1], (0, e_pad - e_rows),
                    constant_values=num_nodes)
    nf = jnp.concatenate([a_node_features, b_node_features], axis=0)
    ef = jnp.concatenate(
        [a_edge_features, b_edge_features,
         jnp.full((2 * nc, 1), -1000.0, jnp.float32)], axis=0)
    ef = _pad_rows(ef, e_pad)
    idx_cls = jnp.pad(jnp.concatenate([pb0, pb1]), (0, cls_pad - 2 * nc))
    zeros_half = jnp.zeros((num_nodes // 2, HN), jnp.float32)

    wm_h, wm_e = Wm[:HN], Wm[HN:]
    wu_h, wu_a = Wu[:HN], Wu[HN:]
    wue_s, wue_d, wue_e = Wue[:HN], Wue[HN:2 * HN], Wue[2 * HN:]
    w1a, w1b, w1p = W1[:HN], W1[HN:2 * HN], W1[2 * HN:]
    bn2, bm2, bu2, be2, bue2 = (x.reshape(1, -1) for x in (bn, bm, bu, be, bue))
    b1r, b2r, b3r, b4r, b5r, b6r = (x.reshape(1, -1)
                                    for x in (b1, b2, b3, b4, b5, b6))

    h, p = _tc_node_init(nf, Wn, bn2, wm_h)
    e = _tc_edge_init(ef, We, be2)
    for _ in range(NUM_MPN_STEPS):
        g = _sc_gather(p, src, dim=HN)
        msg = _tc_msg(g, e, wm_e, bm2)
        agg = _sc_segment_sum(msg, dst_s, zeros_half, num_nodes)
        h, qs, qd, p = _tc_node_update(h, agg, wu_h, wu_a, bu2,
                                       wue_s, wue_d, wm_h)
        sd = _sc_gather_add2(qs, qd, src, dst_g)
        e = _tc_edge_update(sd, e, wue_e, bue2)

    hh = _sc_gather(h, idx_cls, dim=HN)
    ha, hb = hh[:nc], hh[nc:2 * nc]
    off = ea + eb
    e1 = e[off:off + nc]
    e2 = e[off + nc:off + 2 * nc]
    return _tc_classifier(ha, hb, e1, e2, w1a, w1b, w1p, b1r,
                          W2, b2r, W3, b3r, W4, b4r, W5, b5r, W6, b6r)
